# baseline (speedup 1.0000x reference)
import functools

import jax
import jax.numpy as jnp
from jax import lax
from jax.experimental import pallas as pl
from jax.experimental.pallas import tpu as pltpu
from jax.experimental.pallas import tpu_sc as plsc

N = 2048
D = 1024
K = 8192

_info = plsc.get_sparse_core_info()
_NC = _info.num_cores
_NS = _info.num_subcores
_NW = _NC * _NS
_BPW = N // _NW
_NCH = 2
_HPW = _BPW // _NCH


def _sc_gather(codebook, target):
  mesh = plsc.VectorSubcoreMesh(core_axis_name="c", subcore_axis_name="s")

  @functools.partial(
      pl.kernel,
      mesh=mesh,
      out_type=jax.ShapeDtypeStruct((N, D), jnp.float32),
      scratch_types=(
          [pltpu.VMEM((_HPW,), jnp.int32) for _ in range(_NCH)]
          + [pltpu.VMEM((_HPW, D), jnp.float32) for _ in range(_NCH)]
          + [pltpu.SemaphoreType.DMA for _ in range(_NCH)]
      ),
  )
  def k(table_hbm, idx_hbm, out_hbm, *scr):
    idx = scr[:_NCH]
    rows = scr[_NCH:2 * _NCH]
    sems = scr[2 * _NCH:]
    wid = lax.axis_index("s") * _NC + lax.axis_index("c")
    base = wid * _BPW
    for c in range(_NCH):
      pltpu.sync_copy(idx_hbm.at[pl.ds(base + c * _HPW, _HPW)], idx[c])
    cps = [None] * _NCH
    cps[0] = pltpu.async_copy(table_hbm.at[idx[0]], rows[0], sems[0])
    for c in range(_NCH):
      cps[c].wait()
      if c + 1 < _NCH:
        cps[c + 1] = pltpu.async_copy(table_hbm.at[idx[c + 1]],
                                      rows[c + 1], sems[c + 1])
      pltpu.sync_copy(rows[c], out_hbm.at[pl.ds(base + c * _HPW, _HPW)])

  return k(codebook, target)


_BJ = 128


def _loss_body(pred_ref, e_ref, mask_ref, out_ref, p2_ref):
  j = pl.program_id(0)
  nj = pl.num_programs(0)

  @pl.when(j == 0)
  def _():
    p2_ref[...] = jnp.sum(pred_ref[...] * pred_ref[...], axis=1,
                          keepdims=True)
    out_ref[0, 0] = 0.0

  e = e_ref[...]
  g = lax.dot_general(pred_ref[...], e, (((1,), (1,)), ((), ())),
                      preferred_element_type=jnp.float32)
  t2 = jnp.sum(e * e, axis=1)
  d2 = p2_ref[...] + t2[None, :] - 2.0 * g
  d2 = jnp.maximum(d2, 1e-30)
  part = jnp.sum((d2 * lax.rsqrt(d2)) * mask_ref[...])
  acc = out_ref[0, 0] + part

  @pl.when(j < nj - 1)
  def _():
    out_ref[0, 0] = acc

  @pl.when(j == nj - 1)
  def _():
    msum = jnp.sum(mask_ref[...])
    out_ref[0, 0] = acc / (msum * D)


def kernel(pred, target, codebook):
  emb = _sc_gather(codebook, target)
  maskf = (target != -1).astype(jnp.float32).reshape(N, 1)

  out = pl.pallas_call(
      _loss_body,
      grid=(N // _BJ,),
      in_specs=[
          pl.BlockSpec((N, D), lambda j: (0, 0)),
          pl.BlockSpec((_BJ, D), lambda j: (j, 0)),
          pl.BlockSpec((N, 1), lambda j: (0, 0)),
      ],
      out_specs=pl.BlockSpec(memory_space=pltpu.SMEM),
      out_shape=jax.ShapeDtypeStruct((1, 1), jnp.float32),
      scratch_shapes=[pltpu.VMEM((N, 1), jnp.float32)],
  )(pred, emb, maskf)
  return out[0, 0]


# speedup vs baseline: 1.1971x; 1.1971x over previous
import functools

import jax
import jax.numpy as jnp
from jax import lax
from jax.experimental import pallas as pl
from jax.experimental.pallas import tpu as pltpu
from jax.experimental.pallas import tpu_sc as plsc

N = 2048
D = 1024
K = 8192

_info = plsc.get_sparse_core_info()
_NC = _info.num_cores
_NS = _info.num_subcores
_NW = _NC * _NS
_BPW = N // _NW
_NCH = 2
_HPW = _BPW // _NCH


def _sc_gather(codebook, target):
  mesh = plsc.VectorSubcoreMesh(core_axis_name="c", subcore_axis_name="s")

  @functools.partial(
      pl.kernel,
      mesh=mesh,
      out_type=jax.ShapeDtypeStruct((N, D), jnp.float32),
      scratch_types=(
          [pltpu.VMEM((_HPW,), jnp.int32) for _ in range(_NCH)]
          + [pltpu.VMEM((_HPW, D), jnp.float32) for _ in range(_NCH)]
          + [pltpu.SemaphoreType.DMA for _ in range(_NCH)]
      ),
  )
  def k(table_hbm, idx_hbm, out_hbm, *scr):
    idx = scr[:_NCH]
    rows = scr[_NCH:2 * _NCH]
    sems = scr[2 * _NCH:]
    wid = lax.axis_index("s") * _NC + lax.axis_index("c")
    base = wid * _BPW
    for c in range(_NCH):
      pltpu.sync_copy(idx_hbm.at[pl.ds(base + c * _HPW, _HPW)], idx[c])
    cps = [None] * _NCH
    cps[0] = pltpu.async_copy(table_hbm.at[idx[0]], rows[0], sems[0])
    for c in range(_NCH):
      cps[c].wait()
      if c + 1 < _NCH:
        cps[c + 1] = pltpu.async_copy(table_hbm.at[idx[c + 1]],
                                      rows[c + 1], sems[c + 1])
      pltpu.sync_copy(rows[c], out_hbm.at[pl.ds(base + c * _HPW, _HPW)])

  return k(codebook, target)


_BJ = 256


def _loss_body(pred_ref, e_ref, mask_ref, out_ref, p2_ref):
  j = pl.program_id(0)
  nj = pl.num_programs(0)

  @pl.when(j == 0)
  def _():
    p2_ref[...] = jnp.sum(pred_ref[...] * pred_ref[...], axis=1,
                          keepdims=True)
    out_ref[0, 0] = 0.0

  e = e_ref[...]
  g = lax.dot_general(pred_ref[...], e, (((1,), (1,)), ((), ())),
                      preferred_element_type=jnp.float32)
  t2 = jnp.sum(e * e, axis=1)
  d2 = p2_ref[...] + t2[None, :] - 2.0 * g
  d2 = jnp.maximum(d2, 1e-30)
  part = jnp.sum((d2 * lax.rsqrt(d2)) * mask_ref[...])
  acc = out_ref[0, 0] + part

  @pl.when(j < nj - 1)
  def _():
    out_ref[0, 0] = acc

  @pl.when(j == nj - 1)
  def _():
    msum = jnp.sum(mask_ref[...])
    out_ref[0, 0] = acc / (msum * D)


def kernel(pred, target, codebook):
  emb = _sc_gather(codebook, target)
  maskf = (target != -1).astype(jnp.float32).reshape(N, 1)

  out = pl.pallas_call(
      _loss_body,
      grid=(N // _BJ,),
      in_specs=[
          pl.BlockSpec((N, D), lambda j: (0, 0)),
          pl.BlockSpec((_BJ, D), lambda j: (j, 0)),
          pl.BlockSpec((N, 1), lambda j: (0, 0)),
      ],
      out_specs=pl.BlockSpec(memory_space=pltpu.SMEM),
      out_shape=jax.ShapeDtypeStruct((1, 1), jnp.float32),
      scratch_shapes=[pltpu.VMEM((N, 1), jnp.float32)],
  )(pred, emb, maskf)
  return out[0, 0]
